# baseline (speedup 1.0000x reference)
import jax
import jax.numpy as jnp
from jax import lax
from jax.experimental import pallas as pl
from jax.experimental.pallas import tpu as pltpu
from jax.experimental.pallas import tpu_sc as plsc

N = 5000
PADN = 5120
W = 16
C = PADN // W
SLICES = C // 16
SCORE_TH = 0.05
NMS_TH = 0.5
MAX_DET = 100
NEG_INF = float("-inf")

_GATHER_DNUMS = lax.GatherDimensionNumbers(
    offset_dims=(), collapsed_slice_dims=(0,), start_index_map=(0,))


def _shuffle16(x, idx):
    return lax.gather(x, idx[:, None], _GATHER_DNUMS, (1,),
                      mode=lax.GatherScatterMode.PROMISE_IN_BOUNDS)


def _lex_gt(xv, xi, yv, yi):
    return (xv > yv) | ((xv == yv) & (xi < yi))


def _merge2(a1v, a1i, a2v, a2i, b1v, b1i, b2v, b2i):
    takeb = _lex_gt(b1v, b1i, a1v, a1i)
    n1v = jnp.where(takeb, b1v, a1v)
    n1i = jnp.where(takeb, b1i, a1i)
    l1v = jnp.where(takeb, a1v, b1v)
    l1i = jnp.where(takeb, a1i, b1i)
    w2v = jnp.where(takeb, b2v, a2v)
    w2i = jnp.where(takeb, b2i, a2i)
    take2 = _lex_gt(w2v, w2i, l1v, l1i)
    n2v = jnp.where(take2, w2v, l1v)
    n2i = jnp.where(take2, w2i, l1i)
    return n1v, n1i, n2v, n2i


def _body(x1_hbm, y1_hbm, x2_hbm, y2_hbm, s_hbm, out_hbm,
          x1f, y1f, x2f, y2f,
          x1c, y1c, x2c, y2c, sc_, areac,
          pub, lpub, win_l, outbuf,
          sh_pub, sh_win):
    cid = lax.axis_index("c")
    wid = lax.axis_index("s")

    @pl.when(cid == 0)
    def _run():
        base = wid * C
        lane = lax.broadcasted_iota(jnp.int32, (16,), 0)

        pltpu.sync_copy(x1_hbm, x1f)
        pltpu.sync_copy(y1_hbm, y1f)
        pltpu.sync_copy(x2_hbm, x2f)
        pltpu.sync_copy(y2_hbm, y2f)
        pltpu.sync_copy(x1_hbm.at[pl.ds(base, C)], x1c)
        pltpu.sync_copy(y1_hbm.at[pl.ds(base, C)], y1c)
        pltpu.sync_copy(x2_hbm.at[pl.ds(base, C)], x2c)
        pltpu.sync_copy(y2_hbm.at[pl.ds(base, C)], y2c)
        pltpu.sync_copy(s_hbm.at[pl.ds(base, C)], sc_)

        for j in range(SLICES):
            sl = pl.ds(j * 16, 16)
            v = sc_[sl]
            sc_[sl] = jnp.where(v > SCORE_TH, v, NEG_INF)
            areac[sl] = (x2c[sl] - x1c[sl]) * (y2c[sl] - y1c[sl])

        def round_body(carry):
            i_out, r = carry
            b1 = jnp.full((16,), NEG_INF, jnp.float32)
            j1 = jnp.zeros((16,), jnp.int32)
            b2 = jnp.full((16,), NEG_INF, jnp.float32)
            j2 = jnp.zeros((16,), jnp.int32)
            for j in range(SLICES):
                v = sc_[pl.ds(j * 16, 16)]
                m1 = v > b1
                m2 = v > b2
                b2n = jnp.where(m1, b1, jnp.where(m2, v, b2))
                j2n = jnp.where(m1, j1, jnp.where(m2, j, j2))
                b1 = jnp.where(m1, v, b1)
                j1 = jnp.where(m1, j, j1)
                b2, j2 = b2n, j2n
            k1 = j1 * 16 + lane + base
            k2 = j2 * 16 + lane + base
            for st in (8, 4, 2, 1):
                perm = lane ^ st
                o1v = _shuffle16(b1, perm)
                o1i = _shuffle16(k1, perm)
                o2v = _shuffle16(b2, perm)
                o2i = _shuffle16(k2, perm)
                b1, k1, b2, k2 = _merge2(b1, k1, b2, k2, o1v, o1i, o2v, o2i)

            pub[pl.ds(0, 16)] = b1
            pub[pl.ds(16, 16)] = plsc.bitcast(k1, jnp.float32)
            pub[pl.ds(32, 16)] = b2
            pub[pl.ds(48, 16)] = plsc.bitcast(k2, jnp.float32)
            pltpu.sync_copy(pub, sh_pub.at[pl.ds(wid * 64, 64)])
            plsc.subcore_barrier()

            @pl.when(wid == 0)
            def _reduce():
                pltpu.sync_copy(sh_pub, lpub)
                g1v = lpub[pl.ds(0, 16)]
                g1i = plsc.bitcast(lpub[pl.ds(16, 16)], jnp.int32)
                g2v = lpub[pl.ds(32, 16)]
                g2i = plsc.bitcast(lpub[pl.ds(48, 16)], jnp.int32)
                for j in range(1, W):
                    o1v = lpub[pl.ds(j * 64, 16)]
                    o1i = plsc.bitcast(lpub[pl.ds(j * 64 + 16, 16)], jnp.int32)
                    o2v = lpub[pl.ds(j * 64 + 32, 16)]
                    o2i = plsc.bitcast(lpub[pl.ds(j * 64 + 48, 16)], jnp.int32)
                    g1v, g1i, g2v, g2i = _merge2(
                        g1v, g1i, g2v, g2i, o1v, o1i, o2v, o2i)
                pub[pl.ds(0, 16)] = g1v
                pub[pl.ds(16, 16)] = plsc.bitcast(g1i, jnp.float32)
                pub[pl.ds(32, 16)] = g2v
                pub[pl.ds(48, 16)] = plsc.bitcast(g2i, jnp.float32)
                pltpu.sync_copy(pub, sh_win)

            plsc.subcore_barrier()
            pltpu.sync_copy(sh_win, win_l)
            wv1 = win_l[pl.ds(0, 16)]
            wg1 = plsc.bitcast(win_l[pl.ds(16, 16)], jnp.int32)
            wv2 = win_l[pl.ds(32, 16)]
            wg2 = plsc.bitcast(win_l[pl.ds(48, 16)], jnp.int32)

            ax1 = plsc.load_gather(x1f, [wg1])
            ay1 = plsc.load_gather(y1f, [wg1])
            ax2 = plsc.load_gather(x2f, [wg1])
            ay2 = plsc.load_gather(y2f, [wg1])
            bx1 = plsc.load_gather(x1f, [wg2])
            by1 = plsc.load_gather(y1f, [wg2])
            bx2 = plsc.load_gather(x2f, [wg2])
            by2 = plsc.load_gather(y2f, [wg2])
            a1area = (ax2 - ax1) * (ay2 - ay1)
            b1area = (bx2 - bx1) * (by2 - by1)
            px1 = jnp.maximum(ax1, bx1)
            py1 = jnp.maximum(ay1, by1)
            px2 = jnp.minimum(ax2, bx2)
            py2 = jnp.minimum(ay2, by2)
            pinter = (jnp.maximum(px2 - px1, 0.0)
                      * jnp.maximum(py2 - py1, 0.0))
            piou = pinter / (a1area + b1area - pinter + 1e-9)
            acc2v = ~(piou > NMS_TH)
            acc2 = acc2v.astype(jnp.int32)[0] > 0

            for j in range(SLICES):
                sl = pl.ds(j * 16, 16)
                cx1 = x1c[sl]
                cy1 = y1c[sl]
                cx2 = x2c[sl]
                cy2 = y2c[sl]
                car = areac[sl]
                gvec = jnp.full((16,), base + j * 16) + lane
                i1x = (jnp.maximum(jnp.minimum(ax2, cx2)
                                   - jnp.maximum(ax1, cx1), 0.0)
                       * jnp.maximum(jnp.minimum(ay2, cy2)
                                     - jnp.maximum(ay1, cy1), 0.0))
                iou1 = i1x / (a1area + car - i1x + 1e-9)
                sup = (iou1 > NMS_TH) | (gvec == wg1)
                i2x = (jnp.maximum(jnp.minimum(bx2, cx2)
                                   - jnp.maximum(bx1, cx1), 0.0)
                       * jnp.maximum(jnp.minimum(by2, cy2)
                                     - jnp.maximum(by1, cy1), 0.0))
                iou2 = i2x / (b1area + car - i2x + 1e-9)
                sup2 = ((iou2 > NMS_TH) | (gvec == wg2)) & acc2v
                sv = sc_[sl]
                sc_[sl] = jnp.where(sup | sup2, NEG_INF, sv)

            @pl.when(wid == 0)
            def _emit():
                lane_ = lane
                v1ok = wv1 > NEG_INF
                row1 = jnp.where(lane_ == 0, ax1,
                       jnp.where(lane_ == 1, ay1,
                       jnp.where(lane_ == 2, ax2,
                       jnp.where(lane_ == 3, ay2,
                       jnp.where(lane_ == 4, wv1, 0.0)))))
                row1 = jnp.where(v1ok, row1, 0.0)
                outbuf[...] = row1
                pltpu.sync_copy(outbuf, out_hbm.at[i_out])

                @pl.when(acc2 & (i_out + 1 < MAX_DET))
                def _emit2():
                    v2ok = wv2 > NEG_INF
                    row2 = jnp.where(lane_ == 0, bx1,
                           jnp.where(lane_ == 1, by1,
                           jnp.where(lane_ == 2, bx2,
                           jnp.where(lane_ == 3, by2,
                           jnp.where(lane_ == 4, wv2, 0.0)))))
                    row2 = jnp.where(v2ok, row2, 0.0)
                    outbuf[...] = row2
                    pltpu.sync_copy(outbuf, out_hbm.at[i_out + 1])

            i_next = i_out + jnp.where(acc2, 2, 1)
            return (i_next, r + 1)

        def cond(carry):
            i_out, r = carry
            return (i_out < MAX_DET) & (r < MAX_DET)

        lax.while_loop(cond, round_body, (jnp.int32(0), jnp.int32(0)))


def kernel(boxes, scores):
    pad = PADN - N
    bpad = jnp.pad(boxes, ((0, pad), (0, 0)))
    x1 = bpad[:, 0]
    y1 = bpad[:, 1]
    x2 = bpad[:, 2]
    y2 = bpad[:, 3]
    sp = jnp.pad(scores, (0, pad))
    mesh = plsc.VectorSubcoreMesh(core_axis_name="c", subcore_axis_name="s",
                                  num_cores=2, num_subcores=16)
    f = pl.kernel(
        _body,
        out_type=jax.ShapeDtypeStruct((MAX_DET, 16), jnp.float32),
        mesh=mesh,
        compiler_params=pltpu.CompilerParams(needs_layout_passes=False),
        scratch_types=[
            pltpu.VMEM((PADN,), jnp.float32),
            pltpu.VMEM((PADN,), jnp.float32),
            pltpu.VMEM((PADN,), jnp.float32),
            pltpu.VMEM((PADN,), jnp.float32),
            pltpu.VMEM((C,), jnp.float32),
            pltpu.VMEM((C,), jnp.float32),
            pltpu.VMEM((C,), jnp.float32),
            pltpu.VMEM((C,), jnp.float32),
            pltpu.VMEM((C,), jnp.float32),
            pltpu.VMEM((C,), jnp.float32),
            pltpu.VMEM((64,), jnp.float32),
            pltpu.VMEM((W * 64,), jnp.float32),
            pltpu.VMEM((64,), jnp.float32),
            pltpu.VMEM((16,), jnp.float32),
            pltpu.VMEM_SHARED((W * 64,), jnp.float32),
            pltpu.VMEM_SHARED((64,), jnp.float32),
        ],
    )
    out = f(x1, y1, x2, y2, sp)
    return out[:, :5]
